# baseline (speedup 1.0000x reference)
import functools

import jax
import jax.numpy as jnp
from jax import lax
from jax.experimental import pallas as pl
from jax.experimental.pallas import tpu as pltpu
from jax.experimental.pallas import tpu_sc as plsc

MAXLEN = 200
EMBED = 64
GROUP = 128
NBUF = 2


def _make_kernel(n_rows, vocab):
    info = plsc.get_sparse_core_info()
    nc, ns = info.num_cores, info.num_subcores
    nw = nc * ns
    per_w = n_rows // nw
    ngroups = per_w // GROUP
    assert per_w % GROUP == 0 and per_w % MAXLEN == 0

    mesh = plsc.VectorSubcoreMesh(core_axis_name="c", subcore_axis_name="s")

    @functools.partial(
        pl.kernel,
        out_type=jax.ShapeDtypeStruct((n_rows, EMBED), jnp.float32),
        mesh=mesh,
        scratch_types=[
            pltpu.VMEM((ngroups, GROUP), jnp.int32),
            pltpu.VMEM((MAXLEN, EMBED), jnp.float32),
            [pltpu.VMEM((GROUP, EMBED), jnp.float32) for _ in range(NBUF)],
            [pltpu.VMEM((GROUP, EMBED), jnp.float32) for _ in range(NBUF)],
            [pltpu.SemaphoreType.DMA for _ in range(NBUF)],
            [pltpu.SemaphoreType.DMA for _ in range(NBUF)],
        ],
        compiler_params=pltpu.CompilerParams(use_tc_tiling_on_sc=False),
    )
    def kern(x_hbm, tok_hbm, pos_hbm, out_hbm,
             idx_v, pos_v, tokbufs, outbufs, gsems, osems):
        wid = lax.axis_index("s") * nc + lax.axis_index("c")
        row0 = wid * per_w
        g0 = wid * ngroups

        pltpu.sync_copy(x_hbm.at[pl.ds(g0, ngroups)], idx_v)
        pltpu.sync_copy(pos_hbm, pos_v)

        def gather(g, b):
            return pltpu.make_async_copy(
                tok_hbm.at[idx_v.at[g]], tokbufs[b], gsems[b])

        def put(g, b):
            return pltpu.make_async_copy(
                outbufs[b], out_hbm.at[pl.ds(row0 + g * GROUP, GROUP)],
                osems[b])

        for b in range(NBUF):
            gather(b, b).start()

        def body(go, _):
            for b in range(NBUF):
                g = go + b
                @pl.when(g >= NBUF)
                def _():
                    put(g - NBUF, b).wait()
                gather(g, b).wait()
                p0 = lax.rem(g * GROUP, MAXLEN)

                def add_row(j, _):
                    p = p0 + j
                    p = jnp.where(p >= MAXLEN, p - MAXLEN, p)
                    for k in range(EMBED // 16):
                        sl = pl.ds(k * 16, 16)
                        outbufs[b][j, sl] = (
                            tokbufs[b][j, sl] + pos_v[p, sl])
                    return 0

                lax.fori_loop(0, GROUP, add_row, 0)

                @pl.when(g + NBUF < ngroups)
                def _():
                    gather(g + NBUF, b).start()
                put(g, b).start()
            return 0

        lax.fori_loop(0, ngroups // NBUF, lambda i, c: body(i * NBUF, c), 0,
                      unroll=False)

        for b in range(NBUF):
            put(ngroups - NBUF + b, b).wait()

    return kern


def kernel(x, token_table, pos_table):
    batch, maxlen = x.shape
    vocab, embed = token_table.shape
    assert maxlen == MAXLEN and embed == EMBED
    n_rows = batch * maxlen
    xf = x.reshape(n_rows // GROUP, GROUP).astype(jnp.int32)
    out = _make_kernel(n_rows, vocab)(xf, token_table, pos_table)
    return out.reshape(batch, maxlen, embed)


# speedup vs baseline: 1.2634x; 1.2634x over previous
import functools

import jax
import jax.numpy as jnp
from jax import lax
from jax.experimental import pallas as pl
from jax.experimental.pallas import tpu as pltpu
from jax.experimental.pallas import tpu_sc as plsc

MAXLEN = 200
EMBED = 64
HALF = MAXLEN // 2
NBUF = 2


def _make_kernel(batch, vocab):
    info = plsc.get_sparse_core_info()
    nc, ns = info.num_cores, info.num_subcores
    nw = nc * ns
    rows_w = batch // nw
    assert batch % nw == 0

    mesh = plsc.VectorSubcoreMesh(core_axis_name="c", subcore_axis_name="s")

    @functools.partial(
        pl.kernel,
        out_type=jax.ShapeDtypeStruct((batch, MAXLEN, EMBED), jnp.float32),
        mesh=mesh,
        scratch_types=[
            pltpu.VMEM((rows_w, 2, HALF), jnp.int32),
            pltpu.VMEM((MAXLEN, EMBED), jnp.float32),
            [pltpu.VMEM((MAXLEN, EMBED), jnp.float32) for _ in range(NBUF)],
            [pltpu.VMEM((MAXLEN, EMBED), jnp.float32) for _ in range(NBUF)],
            [pltpu.SemaphoreType.DMA for _ in range(NBUF)],
            [pltpu.SemaphoreType.DMA for _ in range(NBUF)],
        ],
        compiler_params=pltpu.CompilerParams(use_tc_tiling_on_sc=False),
    )
    def kern(x_hbm, tok_hbm, pos_hbm, out_hbm,
             idx_v, pos_v, tokbufs, outbufs, gsems, osems):
        wid = lax.axis_index("s") * nc + lax.axis_index("c")
        r0 = wid * rows_w

        pltpu.sync_copy(x_hbm.at[pl.ds(r0, rows_w)], idx_v)
        pltpu.sync_copy(pos_hbm, pos_v)

        def gathers(r, b):
            return [
                pltpu.make_async_copy(
                    tok_hbm.at[idx_v.at[r, h]],
                    tokbufs[b].at[pl.ds(h * HALF, HALF)],
                    gsems[b])
                for h in range(2)
            ]

        def put(r, b):
            return pltpu.make_async_copy(
                outbufs[b], out_hbm.at[r0 + r], osems[b])

        for b in range(NBUF):
            for c in gathers(b, b):
                c.start()

        def body(ro, _):
            for b in range(NBUF):
                r = ro + b
                @pl.when(r >= NBUF)
                def _():
                    put(r - NBUF, b).wait()
                for c in gathers(r, b):
                    c.wait()

                def add_row(j, _):
                    for k in range(EMBED // 16):
                        sl = pl.ds(k * 16, 16)
                        outbufs[b][j, sl] = tokbufs[b][j, sl] + pos_v[j, sl]
                    return 0

                lax.fori_loop(0, MAXLEN, add_row, 0)

                @pl.when(r + NBUF < rows_w)
                def _():
                    for c in gathers(r + NBUF, b):
                        c.start()
                put(r, b).start()
            return 0

        lax.fori_loop(0, rows_w // NBUF, lambda i, c: body(i * NBUF, c), 0,
                      unroll=False)

        for b in range(NBUF):
            put(rows_w - NBUF + b, b).wait()

    return kern


def kernel(x, token_table, pos_table):
    batch, maxlen = x.shape
    vocab, embed = token_table.shape
    assert maxlen == MAXLEN and embed == EMBED
    xf = x.reshape(batch, 2, HALF).astype(jnp.int32)
    return _make_kernel(batch, vocab)(xf, token_table, pos_table)
